# baseline (speedup 1.0000x reference)
import functools

import jax
import jax.numpy as jnp
from jax import lax
from jax.experimental import pallas as pl
from jax.experimental.pallas import tpu as pltpu
from jax.experimental.pallas import tpu_sc as plsc

N_BATCH = 16
C_IN = 256
C_OUT = 256
HW = 4096
P_TOTAL = N_BATCH * HW
T = 512
NT = HW // T

SC_CORES = 2
SC_SUBCORES = 16
SC_WORKERS = SC_CORES * SC_SUBCORES
CHUNK = P_TOTAL // SC_WORKERS
N_ANCHOR = 5
LANES = 16


def _sc_mask_kernel(lab_hbm, out_hbm, lab_v, pos_v, neg_v):
    wid = lax.axis_index("s") * SC_CORES + lax.axis_index("c")
    n = wid // (HW // CHUNK)
    off = (wid % (HW // CHUNK)) * CHUNK
    pltpu.sync_copy(lab_hbm.at[n, :, pl.ds(off, CHUNK)], lab_v)

    def body(i, carry):
        base = pl.multiple_of(i * LANES, LANES)
        one = jnp.int32(1)
        zero = jnp.int32(0)
        acc_eq = jnp.zeros((LANES,), jnp.int32)
        acc_ge = jnp.zeros((LANES,), jnp.int32)
        for a_idx in range(N_ANCHOR):
            l = lab_v[a_idx, pl.ds(base, LANES)]
            acc_eq += one - jnp.minimum(jnp.abs(l - one), one)
            acc_ge += one - jnp.minimum(jnp.maximum(-l, zero), one)
        pos_i = jnp.minimum(acc_eq, one)
        neg_i = jnp.minimum(acc_ge, one) * (one - pos_i)
        pos_v[pl.ds(base, LANES)] = pos_i.astype(jnp.float32)
        neg_v[pl.ds(base, LANES)] = neg_i.astype(jnp.float32)
        return carry

    lax.fori_loop(0, CHUNK // LANES, body, 0)
    pltpu.sync_copy(pos_v, out_hbm.at[n, 0, pl.ds(off, CHUNK)])
    pltpu.sync_copy(neg_v, out_hbm.at[n, 1, pl.ds(off, CHUNK)])


def _build_masks(lab):
    mesh = plsc.VectorSubcoreMesh(core_axis_name="c", subcore_axis_name="s")
    return pl.kernel(
        _sc_mask_kernel,
        mesh=mesh,
        out_type=jax.ShapeDtypeStruct((N_BATCH, 2, HW), jnp.float32),
        scratch_types=[
            pltpu.VMEM((N_ANCHOR, CHUNK), jnp.int32),
            pltpu.VMEM((CHUNK,), jnp.float32),
            pltpu.VMEM((CHUNK,), jnp.float32),
        ],
    )(lab)


def _stats_body(x_ref, w_ref, g_ref, be_ref, a_ref, b_ref, s_acc, ss_acc):
    n = pl.program_id(0)
    t = pl.program_id(1)

    @pl.when((n == 0) & (t == 0))
    def _init():
        s_acc[...] = jnp.zeros_like(s_acc)
        ss_acc[...] = jnp.zeros_like(ss_acc)

    y = jnp.dot(w_ref[...], x_ref[0], preferred_element_type=jnp.float32)
    s_acc[...] += y
    ss_acc[...] += y * y

    @pl.when((n == N_BATCH - 1) & (t == NT - 1))
    def _fin():
        inv_p = jnp.float32(1.0 / P_TOTAL)
        mean = jnp.sum(s_acc[...], axis=1, keepdims=True) * inv_p
        ey2 = jnp.sum(ss_acc[...], axis=1, keepdims=True) * inv_p
        var = ey2 - mean * mean
        a = g_ref[...] * lax.rsqrt(var + jnp.float32(1e-5))
        a_ref[...] = a
        b_ref[...] = be_ref[...] - a * mean


def _sums_body(x_ref, w_ref, m_ref, a_ref, b_ref, out_ref,
               sp_acc, ssp_acc, sn_acc, ssn_acc, cnt_acc):
    n = pl.program_id(0)
    t = pl.program_id(1)

    @pl.when((n == 0) & (t == 0))
    def _init():
        sp_acc[...] = jnp.zeros_like(sp_acc)
        ssp_acc[...] = jnp.zeros_like(ssp_acc)
        sn_acc[...] = jnp.zeros_like(sn_acc)
        ssn_acc[...] = jnp.zeros_like(ssn_acc)
        cnt_acc[...] = jnp.zeros_like(cnt_acc)

    y = jnp.dot(w_ref[...], x_ref[0], preferred_element_type=jnp.float32)
    z = jnp.maximum(a_ref[...] * y + b_ref[...], jnp.float32(0.0))
    pm = m_ref[0, 0:1, :]
    nm = m_ref[0, 1:2, :]
    zp = z * pm
    zn = z * nm
    sp_acc[...] += zp
    sn_acc[...] += zn
    ssp_acc[...] += z * zp
    ssn_acc[...] += z * zn
    cnt_acc[...] += m_ref[0]

    @pl.when((n == N_BATCH - 1) & (t == NT - 1))
    def _fin():
        s_pos = jnp.sum(sp_acc[...], axis=1, keepdims=True)
        ss_pos = jnp.sum(ssp_acc[...], axis=1, keepdims=True)
        s_neg = jnp.sum(sn_acc[...], axis=1, keepdims=True)
        ss_neg = jnp.sum(ssn_acc[...], axis=1, keepdims=True)
        n_pos = jnp.sum(cnt_acc[0:1, :])
        n_neg = jnp.sum(cnt_acc[1:2, :])
        mu_p = s_pos / n_pos
        mu_n = s_neg / n_neg
        var_p = (ss_pos - n_pos * mu_p * mu_p) / (n_pos - jnp.float32(1.0))
        var_n = (ss_neg - n_neg * mu_n * mu_n) / (n_neg - jnp.float32(1.0))
        out_ref[...] = jnp.concatenate(
            [mu_p, mu_n, jnp.sqrt(var_p), jnp.sqrt(var_n)], axis=1)


def _tc_stats(x, w, g, be):
    return pl.pallas_call(
        _stats_body,
        grid=(N_BATCH, NT),
        in_specs=[
            pl.BlockSpec((1, C_IN, T), lambda n, t: (n, 0, t)),
            pl.BlockSpec((C_OUT, C_IN), lambda n, t: (0, 0)),
            pl.BlockSpec((C_OUT, 1), lambda n, t: (0, 0)),
            pl.BlockSpec((C_OUT, 1), lambda n, t: (0, 0)),
        ],
        out_specs=[
            pl.BlockSpec((C_OUT, 1), lambda n, t: (0, 0)),
            pl.BlockSpec((C_OUT, 1), lambda n, t: (0, 0)),
        ],
        out_shape=[
            jax.ShapeDtypeStruct((C_OUT, 1), jnp.float32),
            jax.ShapeDtypeStruct((C_OUT, 1), jnp.float32),
        ],
        scratch_shapes=[
            pltpu.VMEM((C_OUT, T), jnp.float32),
            pltpu.VMEM((C_OUT, T), jnp.float32),
        ],
    )(x, w, g, be)


def _tc_sums(x, w, masks, a, b):
    return pl.pallas_call(
        _sums_body,
        grid=(N_BATCH, NT),
        in_specs=[
            pl.BlockSpec((1, C_IN, T), lambda n, t: (n, 0, t)),
            pl.BlockSpec((C_OUT, C_IN), lambda n, t: (0, 0)),
            pl.BlockSpec((1, 2, T), lambda n, t: (n, 0, t)),
            pl.BlockSpec((C_OUT, 1), lambda n, t: (0, 0)),
            pl.BlockSpec((C_OUT, 1), lambda n, t: (0, 0)),
        ],
        out_specs=pl.BlockSpec((C_OUT, 4), lambda n, t: (0, 0)),
        out_shape=jax.ShapeDtypeStruct((C_OUT, 4), jnp.float32),
        scratch_shapes=[
            pltpu.VMEM((C_OUT, T), jnp.float32),
            pltpu.VMEM((C_OUT, T), jnp.float32),
            pltpu.VMEM((C_OUT, T), jnp.float32),
            pltpu.VMEM((C_OUT, T), jnp.float32),
            pltpu.VMEM((2, T), jnp.float32),
        ],
    )(x, w, masks, a, b)


def kernel(hidden_map, label_cls, W, gamma, beta):
    s = hidden_map.shape
    x = hidden_map.reshape(N_BATCH, C_IN, HW)
    lab = label_cls.astype(jnp.int32).reshape(N_BATCH, N_ANCHOR, HW)
    masks = _build_masks(lab)
    g = gamma.reshape(C_OUT, 1)
    be = beta.reshape(C_OUT, 1)
    a, b = _tc_stats(x, W, g, be)
    out = _tc_sums(x, W, masks, a, b)
    latents = out.T.reshape(1, 4 * C_OUT, 1, 1)
    return (latents, 0.0, s)


# speedup vs baseline: 1.7167x; 1.7167x over previous
import jax
import jax.numpy as jnp
from jax import lax
from jax.experimental import pallas as pl
from jax.experimental.pallas import tpu as pltpu
from jax.experimental.pallas import tpu_sc as plsc

N_BATCH = 16
C_IN = 256
C_OUT = 256
HW = 4096
P_TOTAL = N_BATCH * HW
RT = 512

SC_CORES = 2
SC_SUBCORES = 16
SC_WORKERS = SC_CORES * SC_SUBCORES
CHUNK = P_TOTAL // SC_WORKERS
N_ANCHOR = 5
LANES = 16


def _sc_mask_kernel(lab_hbm, out_hbm, lab_v, pos_v, neg_v):
    wid = lax.axis_index("s") * SC_CORES + lax.axis_index("c")
    n = wid // (HW // CHUNK)
    off = (wid % (HW // CHUNK)) * CHUNK
    pltpu.sync_copy(lab_hbm.at[n, :, pl.ds(off, CHUNK)], lab_v)

    def body(i, carry):
        base = pl.multiple_of(i * LANES, LANES)
        one = jnp.int32(1)
        zero = jnp.int32(0)
        acc_eq = jnp.zeros((LANES,), jnp.int32)
        acc_ge = jnp.zeros((LANES,), jnp.int32)
        for a_idx in range(N_ANCHOR):
            l = lab_v[a_idx, pl.ds(base, LANES)]
            acc_eq += one - jnp.minimum(jnp.abs(l - one), one)
            acc_ge += one - jnp.minimum(jnp.maximum(-l, zero), one)
        pos_i = jnp.minimum(acc_eq, one)
        neg_i = jnp.minimum(acc_ge, one) * (one - pos_i)
        pos_v[pl.ds(base, LANES)] = pos_i.astype(jnp.float32)
        neg_v[pl.ds(base, LANES)] = neg_i.astype(jnp.float32)
        return carry

    lax.fori_loop(0, CHUNK // LANES, body, 0)
    pltpu.sync_copy(pos_v, out_hbm.at[n, 0, pl.ds(off, CHUNK)])
    pltpu.sync_copy(neg_v, out_hbm.at[n, 1, pl.ds(off, CHUNK)])


def _build_masks(lab):
    mesh = plsc.VectorSubcoreMesh(core_axis_name="c", subcore_axis_name="s")
    return pl.kernel(
        _sc_mask_kernel,
        mesh=mesh,
        out_type=jax.ShapeDtypeStruct((N_BATCH, 2, HW), jnp.float32),
        scratch_types=[
            pltpu.VMEM((N_ANCHOR, CHUNK), jnp.int32),
            pltpu.VMEM((CHUNK,), jnp.float32),
            pltpu.VMEM((CHUNK,), jnp.float32),
        ],
    )(lab)


def _fold(v):
    v = v[:, :2048] + v[:, 2048:]
    v = v[:, :1024] + v[:, 1024:]
    return v[:, :512] + v[:, 512:]


def _tc_body(x_ref, w_ref, m_ref, g_ref, be_ref, out_ref,
             xbf_s, s_acc, ss_acc, sp_acc, ssp_acc, sn_acc, ssn_acc,
             cnt_acc, a_s, b_s):
    ph = pl.program_id(0)
    n = pl.program_id(1)

    @pl.when((ph == 0) & (n == 0))
    def _init():
        s_acc[...] = jnp.zeros_like(s_acc)
        ss_acc[...] = jnp.zeros_like(ss_acc)
        sp_acc[...] = jnp.zeros_like(sp_acc)
        ssp_acc[...] = jnp.zeros_like(ssp_acc)
        sn_acc[...] = jnp.zeros_like(sn_acc)
        ssn_acc[...] = jnp.zeros_like(ssn_acc)
        cnt_acc[...] = jnp.zeros_like(cnt_acc)

    @pl.when(ph == 0)
    def _phase0():
        xbf3 = x_ref[...].astype(jnp.bfloat16)
        xbf_s[pl.ds(n, 1)] = xbf3
        y = jnp.dot(w_ref[...], xbf3[0], preferred_element_type=jnp.float32)
        s_acc[...] += _fold(y)
        ss_acc[...] += _fold(y * y)

    @pl.when((ph == 0) & (n == N_BATCH - 1))
    def _mid():
        inv_p = jnp.float32(1.0 / P_TOTAL)
        mean = jnp.sum(s_acc[...], axis=1, keepdims=True) * inv_p
        ey2 = jnp.sum(ss_acc[...], axis=1, keepdims=True) * inv_p
        var = ey2 - mean * mean
        a = g_ref[...] * lax.rsqrt(var + jnp.float32(1e-5))
        a_s[...] = a
        b_s[...] = be_ref[...] - a * mean

    @pl.when(ph == 1)
    def _phase1():
        xv = xbf_s[n]
        y = jnp.dot(w_ref[...], xv, preferred_element_type=jnp.float32)
        z = jnp.maximum(a_s[...] * y + b_s[...], jnp.float32(0.0))
        pm = m_ref[0, 0:1, :]
        nm = m_ref[0, 1:2, :]
        zp = z * pm
        zn = z * nm
        sp_acc[...] += _fold(zp)
        ssp_acc[...] += _fold(z * zp)
        sn_acc[...] += _fold(zn)
        ssn_acc[...] += _fold(z * zn)
        cnt_acc[...] += _fold(m_ref[0])

    @pl.when((ph == 1) & (n == N_BATCH - 1))
    def _fin():
        s_pos = jnp.sum(sp_acc[...], axis=1, keepdims=True)
        ss_pos = jnp.sum(ssp_acc[...], axis=1, keepdims=True)
        s_neg = jnp.sum(sn_acc[...], axis=1, keepdims=True)
        ss_neg = jnp.sum(ssn_acc[...], axis=1, keepdims=True)
        n_pos = jnp.sum(cnt_acc[0:1, :])
        n_neg = jnp.sum(cnt_acc[1:2, :])
        mu_p = s_pos / n_pos
        mu_n = s_neg / n_neg
        var_p = (ss_pos - n_pos * mu_p * mu_p) / (n_pos - jnp.float32(1.0))
        var_n = (ss_neg - n_neg * mu_n * mu_n) / (n_neg - jnp.float32(1.0))
        out_ref[...] = jnp.concatenate(
            [mu_p, mu_n, jnp.sqrt(var_p), jnp.sqrt(var_n)], axis=1)


def _tc_call(x, wbf, masks, g, be):
    return pl.pallas_call(
        _tc_body,
        grid=(2, N_BATCH),
        in_specs=[
            pl.BlockSpec((1, C_IN, HW), lambda ph, n: (n * (1 - ph), 0, 0)),
            pl.BlockSpec((C_OUT, C_IN), lambda ph, n: (0, 0)),
            pl.BlockSpec((1, 2, HW), lambda ph, n: (n, 0, 0)),
            pl.BlockSpec((C_OUT, 1), lambda ph, n: (0, 0)),
            pl.BlockSpec((C_OUT, 1), lambda ph, n: (0, 0)),
        ],
        out_specs=pl.BlockSpec((C_OUT, 4), lambda ph, n: (0, 0)),
        out_shape=jax.ShapeDtypeStruct((C_OUT, 4), jnp.float32),
        scratch_shapes=[
            pltpu.VMEM((N_BATCH, C_IN, HW), jnp.bfloat16),
            pltpu.VMEM((C_OUT, RT), jnp.float32),
            pltpu.VMEM((C_OUT, RT), jnp.float32),
            pltpu.VMEM((C_OUT, RT), jnp.float32),
            pltpu.VMEM((C_OUT, RT), jnp.float32),
            pltpu.VMEM((C_OUT, RT), jnp.float32),
            pltpu.VMEM((C_OUT, RT), jnp.float32),
            pltpu.VMEM((2, RT), jnp.float32),
            pltpu.VMEM((C_OUT, 1), jnp.float32),
            pltpu.VMEM((C_OUT, 1), jnp.float32),
        ],
    )(x, wbf, masks, g, be)


def kernel(hidden_map, label_cls, W, gamma, beta):
    s = hidden_map.shape
    x = hidden_map.reshape(N_BATCH, C_IN, HW)
    lab = label_cls.astype(jnp.int32).reshape(N_BATCH, N_ANCHOR, HW)
    masks = _build_masks(lab)
    g = gamma.reshape(C_OUT, 1)
    be = beta.reshape(C_OUT, 1)
    wbf = W.astype(jnp.bfloat16)
    out = _tc_call(x, wbf, masks, g, be)
    latents = out.T.reshape(1, 4 * C_OUT, 1, 1)
    return (latents, 0.0, s)


# speedup vs baseline: 2.4338x; 1.4177x over previous
import jax
import jax.numpy as jnp
from jax import lax
from jax.experimental import pallas as pl
from jax.experimental.pallas import tpu as pltpu
from jax.experimental.pallas import tpu_sc as plsc

N_BATCH = 16
C_IN = 256
C_OUT = 256
HW = 4096
P_TOTAL = N_BATCH * HW
TP = 2048
NTP = P_TOTAL // TP
MROWS = 8

SC_CORES = 2
SC_SUBCORES = 16
N_ANCHOR = 5
LANES = 16


def _sc_mask_kernel(lab_hbm, out_hbm, lab_v, out_v):
    wid = lax.axis_index("s") * SC_CORES + lax.axis_index("c")
    n = wid // (HW // TP)
    off = (wid % (HW // TP)) * TP
    pltpu.sync_copy(lab_hbm.at[n, :, pl.ds(off, TP)], lab_v)

    def body(i, carry):
        base = pl.multiple_of(i * LANES, LANES)
        one = jnp.int32(1)
        zero = jnp.int32(0)
        acc_eq = jnp.zeros((LANES,), jnp.int32)
        acc_ge = jnp.zeros((LANES,), jnp.int32)
        for a_idx in range(N_ANCHOR):
            l = lab_v[a_idx, pl.ds(base, LANES)]
            acc_eq += one - jnp.minimum(jnp.abs(l - one), one)
            acc_ge += one - jnp.minimum(jnp.maximum(-l, zero), one)
        pos_i = jnp.minimum(acc_eq, one)
        neg_i = jnp.minimum(acc_ge, one) * (one - pos_i)
        zf = jnp.zeros((LANES,), jnp.float32)
        out_v[0, pl.ds(base, LANES)] = pos_i.astype(jnp.float32)
        out_v[1, pl.ds(base, LANES)] = neg_i.astype(jnp.float32)
        for r in range(2, MROWS):
            out_v[r, pl.ds(base, LANES)] = zf
        return carry

    lax.fori_loop(0, TP // LANES, body, 0)
    pltpu.sync_copy(out_v, out_hbm.at[wid])


def _build_masks(lab):
    mesh = plsc.VectorSubcoreMesh(core_axis_name="c", subcore_axis_name="s")
    return pl.kernel(
        _sc_mask_kernel,
        mesh=mesh,
        out_type=jax.ShapeDtypeStruct((NTP, MROWS, TP), jnp.float32),
        scratch_types=[
            pltpu.VMEM((N_ANCHOR, TP), jnp.int32),
            pltpu.VMEM((MROWS, TP), jnp.float32),
        ],
    )(lab)


def _fold8(v):
    v = v[:, :1024] + v[:, 1024:]
    v = v[:, :512] + v[:, 512:]
    return v[:, :256] + v[:, 256:]


def _dot_t(a, b):
    return lax.dot_general(a, b, (((1,), (1,)), ((), ())),
                           preferred_element_type=jnp.float32)


def _tc_body(x_ref, w_ref, m_ref, gb_ref, out_ref,
             xbf_s, asum, asq, macc, mqacc, cnt, a_s, b_s):
    ph = pl.program_id(0)
    t = pl.program_id(1)

    @pl.when((ph == 0) & (t == 0))
    def _init():
        asum[...] = jnp.zeros_like(asum)
        asq[...] = jnp.zeros_like(asq)
        macc[...] = jnp.zeros_like(macc)
        mqacc[...] = jnp.zeros_like(mqacc)
        cnt[...] = jnp.zeros_like(cnt)

    wbf = w_ref[...].astype(jnp.bfloat16)

    @pl.when(ph == 0)
    def _phase0():
        xbf = x_ref[...].astype(jnp.bfloat16)
        xbf_s[pl.ds(t * TP, TP), :] = xbf
        y = _dot_t(xbf, wbf)
        o1 = jnp.ones((1, TP), jnp.float32)
        asum[...] += jnp.dot(o1, y, preferred_element_type=jnp.float32)
        asq[...] += jnp.dot(o1, y * y, preferred_element_type=jnp.float32)

    @pl.when((ph == 0) & (t == NTP - 1))
    def _mid():
        inv_p = jnp.float32(1.0 / P_TOTAL)
        mean = asum[...] * inv_p
        var = asq[...] * inv_p - mean * mean
        a = gb_ref[0:1, :] * lax.rsqrt(var + jnp.float32(1e-5))
        a_s[...] = a
        b_s[...] = gb_ref[1:2, :] - a * mean

    @pl.when(ph == 1)
    def _phase1():
        xv = xbf_s[pl.ds(t * TP, TP), :]
        y = _dot_t(xv, wbf)
        z = jnp.maximum(a_s[...] * y + b_s[...], jnp.float32(0.0))
        m2 = m_ref[0]
        macc[...] += jnp.dot(m2, z, preferred_element_type=jnp.float32)
        mqacc[...] += jnp.dot(m2, z * z, preferred_element_type=jnp.float32)
        cnt[...] += _fold8(m2)

    @pl.when((ph == 1) & (t == NTP - 1))
    def _fin():
        n_pos = jnp.sum(cnt[0:1, :])
        n_neg = jnp.sum(cnt[1:2, :])
        mu_p = macc[0:1, :] / n_pos
        mu_n = macc[1:2, :] / n_neg
        var_p = (mqacc[0:1, :] - n_pos * mu_p * mu_p) / (n_pos - jnp.float32(1.0))
        var_n = (mqacc[1:2, :] - n_neg * mu_n * mu_n) / (n_neg - jnp.float32(1.0))
        out_ref[...] = jnp.concatenate(
            [mu_p, mu_n, jnp.sqrt(var_p), jnp.sqrt(var_n)], axis=0)


def _tc_call(xt, w, masks, gb):
    return pl.pallas_call(
        _tc_body,
        grid=(2, NTP),
        in_specs=[
            pl.BlockSpec((TP, C_IN), lambda ph, t: (t * (1 - ph), 0)),
            pl.BlockSpec((C_OUT, C_IN), lambda ph, t: (0, 0)),
            pl.BlockSpec((1, MROWS, TP), lambda ph, t: (t, 0, 0)),
            pl.BlockSpec((2, C_OUT), lambda ph, t: (0, 0)),
        ],
        out_specs=pl.BlockSpec((4, C_OUT), lambda ph, t: (0, 0)),
        out_shape=jax.ShapeDtypeStruct((4, C_OUT), jnp.float32),
        scratch_shapes=[
            pltpu.VMEM((P_TOTAL, C_IN), jnp.bfloat16),
            pltpu.VMEM((1, C_OUT), jnp.float32),
            pltpu.VMEM((1, C_OUT), jnp.float32),
            pltpu.VMEM((MROWS, C_OUT), jnp.float32),
            pltpu.VMEM((MROWS, C_OUT), jnp.float32),
            pltpu.VMEM((MROWS, C_OUT), jnp.float32),
            pltpu.VMEM((1, C_OUT), jnp.float32),
            pltpu.VMEM((1, C_OUT), jnp.float32),
        ],
    )(xt, w, masks, gb)


def kernel(hidden_map, label_cls, W, gamma, beta):
    s = hidden_map.shape
    xt = jnp.transpose(hidden_map, (0, 2, 3, 1)).reshape(P_TOTAL, C_IN)
    lab = label_cls.astype(jnp.int32).reshape(N_BATCH, N_ANCHOR, HW)
    masks = _build_masks(lab)
    gb = jnp.stack([gamma, beta])
    out = _tc_call(xt, W, masks, gb)
    latents = out.reshape(1, 4 * C_OUT, 1, 1)
    return (latents, 0.0, s)


# speedup vs baseline: 3.2804x; 1.3478x over previous
import jax
import jax.numpy as jnp
from jax import lax
from jax.experimental import pallas as pl
from jax.experimental.pallas import tpu as pltpu
from jax.experimental.pallas import tpu_sc as plsc

N_BATCH = 16
C_IN = 256
C_OUT = 256
HW = 4096
P_TOTAL = N_BATCH * HW
TP = 4096
NTP = P_TOTAL // TP
MROWS = 8
SC_CHUNK = 2048

SC_CORES = 2
SC_SUBCORES = 16
N_ANCHOR = 5
LANES = 16


def _sc_mask_kernel(lab_hbm, out_hbm, lab_v, out_v):
    wid = lax.axis_index("s") * SC_CORES + lax.axis_index("c")
    n = wid // (HW // SC_CHUNK)
    off = (wid % (HW // SC_CHUNK)) * SC_CHUNK
    pltpu.sync_copy(lab_hbm.at[n, :, pl.ds(off, SC_CHUNK)], lab_v)

    def body(i, carry):
        base = pl.multiple_of(i * LANES, LANES)
        one = jnp.int32(1)
        zero = jnp.int32(0)
        acc_eq = jnp.zeros((LANES,), jnp.int32)
        acc_ge = jnp.zeros((LANES,), jnp.int32)
        for a_idx in range(N_ANCHOR):
            l = lab_v[a_idx, pl.ds(base, LANES)]
            acc_eq += one - jnp.minimum(jnp.abs(l - one), one)
            acc_ge += one - jnp.minimum(jnp.maximum(-l, zero), one)
        pos_i = jnp.minimum(acc_eq, one)
        neg_i = jnp.minimum(acc_ge, one) * (one - pos_i)
        zf = jnp.zeros((LANES,), jnp.float32)
        out_v[0, pl.ds(base, LANES)] = pos_i.astype(jnp.float32)
        out_v[1, pl.ds(base, LANES)] = neg_i.astype(jnp.float32)
        for r in range(2, MROWS):
            out_v[r, pl.ds(base, LANES)] = zf
        return carry

    lax.fori_loop(0, SC_CHUNK // LANES, body, 0)
    t = wid // (TP // SC_CHUNK)
    col = (wid % (TP // SC_CHUNK)) * SC_CHUNK
    pltpu.sync_copy(out_v, out_hbm.at[t, :, pl.ds(col, SC_CHUNK)])


def _build_masks(lab):
    mesh = plsc.VectorSubcoreMesh(core_axis_name="c", subcore_axis_name="s")
    return pl.kernel(
        _sc_mask_kernel,
        mesh=mesh,
        out_type=jax.ShapeDtypeStruct((NTP, MROWS, TP), jnp.float32),
        scratch_types=[
            pltpu.VMEM((N_ANCHOR, SC_CHUNK), jnp.int32),
            pltpu.VMEM((MROWS, SC_CHUNK), jnp.float32),
        ],
    )(lab)


def _fold8(v):
    w = v.shape[1]
    while w > 256:
        w //= 2
        v = v[:, :w] + v[:, w:]
    return v


def _dot_t(a, b):
    return lax.dot_general(a, b, (((1,), (1,)), ((), ())),
                           preferred_element_type=jnp.float32)


def _tc_body(x_ref, w_ref, m_ref, gb_ref, out_ref,
             ybf_s, asum, asq, macc, cnt, a_s, b_s):
    ph = pl.program_id(0)
    t = pl.program_id(1)

    @pl.when((ph == 0) & (t == 0))
    def _init():
        asum[...] = jnp.zeros_like(asum)
        asq[...] = jnp.zeros_like(asq)
        macc[...] = jnp.zeros_like(macc)
        cnt[...] = jnp.zeros_like(cnt)

    @pl.when(ph == 0)
    def _phase0():
        xbf = x_ref[...].astype(jnp.bfloat16)
        wbf = w_ref[...].astype(jnp.bfloat16)
        ybf = _dot_t(xbf, wbf).astype(jnp.bfloat16)
        ybf_s[pl.ds(t * TP, TP), :] = ybf
        o1 = jnp.ones((1, TP), jnp.bfloat16)
        asum[...] += jnp.dot(o1, ybf, preferred_element_type=jnp.float32)
        asq[...] += jnp.dot(o1, ybf * ybf, preferred_element_type=jnp.float32)

    @pl.when((ph == 0) & (t == NTP - 1))
    def _mid():
        inv_p = jnp.float32(1.0 / P_TOTAL)
        mean = asum[...] * inv_p
        var = asq[...] * inv_p - mean * mean
        a = gb_ref[0:1, :] * lax.rsqrt(var + jnp.float32(1e-5))
        a_s[...] = a
        b_s[...] = gb_ref[1:2, :] - a * mean

    @pl.when(ph == 1)
    def _phase1():
        yv = ybf_s[pl.ds(t * TP, TP), :]
        z = jnp.maximum(a_s[...] * yv + b_s[...], jnp.float32(0.0))
        zz = jnp.concatenate([z, z * z], axis=1).astype(jnp.bfloat16)
        m2 = m_ref[0]
        macc[...] += jnp.dot(m2.astype(jnp.bfloat16), zz,
                             preferred_element_type=jnp.float32)
        cnt[...] += _fold8(m2)

    @pl.when((ph == 1) & (t == NTP - 1))
    def _fin():
        n_pos = jnp.sum(cnt[0:1, :])
        n_neg = jnp.sum(cnt[1:2, :])
        mu_p = macc[0:1, :C_OUT] / n_pos
        mu_n = macc[1:2, :C_OUT] / n_neg
        ss_p = macc[0:1, C_OUT:]
        ss_n = macc[1:2, C_OUT:]
        var_p = (ss_p - n_pos * mu_p * mu_p) / (n_pos - jnp.float32(1.0))
        var_n = (ss_n - n_neg * mu_n * mu_n) / (n_neg - jnp.float32(1.0))
        out_ref[...] = jnp.concatenate(
            [mu_p, mu_n, jnp.sqrt(var_p), jnp.sqrt(var_n)], axis=0)


def _tc_call(xt, w, masks, gb):
    return pl.pallas_call(
        _tc_body,
        grid=(2, NTP),
        in_specs=[
            pl.BlockSpec((TP, C_IN), lambda ph, t: (t * (1 - ph), 0)),
            pl.BlockSpec((C_OUT, C_IN), lambda ph, t: (0, 0)),
            pl.BlockSpec((1, MROWS, TP), lambda ph, t: (t, 0, 0)),
            pl.BlockSpec((2, C_OUT), lambda ph, t: (0, 0)),
        ],
        out_specs=pl.BlockSpec((4, C_OUT), lambda ph, t: (0, 0)),
        out_shape=jax.ShapeDtypeStruct((4, C_OUT), jnp.float32),
        scratch_shapes=[
            pltpu.VMEM((P_TOTAL, C_OUT), jnp.bfloat16),
            pltpu.VMEM((1, C_OUT), jnp.float32),
            pltpu.VMEM((1, C_OUT), jnp.float32),
            pltpu.VMEM((MROWS, 2 * C_OUT), jnp.float32),
            pltpu.VMEM((MROWS, C_OUT), jnp.float32),
            pltpu.VMEM((1, C_OUT), jnp.float32),
            pltpu.VMEM((1, C_OUT), jnp.float32),
        ],
    )(xt, w, masks, gb)


def kernel(hidden_map, label_cls, W, gamma, beta):
    s = hidden_map.shape
    xt = jnp.transpose(hidden_map, (0, 2, 3, 1)).reshape(P_TOTAL, C_IN)
    lab = label_cls.astype(jnp.int32).reshape(N_BATCH, N_ANCHOR, HW)
    masks = _build_masks(lab)
    gb = jnp.stack([gamma, beta])
    out = _tc_call(xt, W, masks, gb)
    latents = out.reshape(1, 4 * C_OUT, 1, 1)
    return (latents, 0.0, s)


# speedup vs baseline: 3.6581x; 1.1151x over previous
import jax
import jax.numpy as jnp
from jax import lax
from jax.experimental import pallas as pl
from jax.experimental.pallas import tpu as pltpu
from jax.experimental.pallas import tpu_sc as plsc

N_BATCH = 16
C_IN = 256
C_OUT = 256
HW = 4096
P_TOTAL = N_BATCH * HW
TP = 8192
NTP = P_TOTAL // TP
MROWS = 8
SC_CHUNK = 2048

SC_CORES = 2
SC_SUBCORES = 16
N_ANCHOR = 5
LANES = 16


def _sc_mask_kernel(lab_hbm, out_hbm, lab_v, out_v):
    wid = lax.axis_index("s") * SC_CORES + lax.axis_index("c")
    n = wid // (HW // SC_CHUNK)
    off = (wid % (HW // SC_CHUNK)) * SC_CHUNK
    pltpu.sync_copy(lab_hbm.at[n, :, pl.ds(off, SC_CHUNK)], lab_v)

    def body(i, carry):
        base = pl.multiple_of(i * LANES, LANES)
        one = jnp.int32(1)
        zero = jnp.int32(0)
        acc_eq = jnp.zeros((LANES,), jnp.int32)
        acc_ge = jnp.zeros((LANES,), jnp.int32)
        for a_idx in range(N_ANCHOR):
            l = lab_v[a_idx, pl.ds(base, LANES)]
            acc_eq += one - jnp.minimum(jnp.abs(l - one), one)
            acc_ge += one - jnp.minimum(jnp.maximum(-l, zero), one)
        pos_i = jnp.minimum(acc_eq, one)
        neg_i = jnp.minimum(acc_ge, one) * (one - pos_i)
        zf = jnp.zeros((LANES,), jnp.float32)
        out_v[0, pl.ds(base, LANES)] = pos_i.astype(jnp.float32)
        out_v[1, pl.ds(base, LANES)] = neg_i.astype(jnp.float32)
        for r in range(2, MROWS):
            out_v[r, pl.ds(base, LANES)] = zf
        return carry

    lax.fori_loop(0, SC_CHUNK // LANES, body, 0)
    t = wid // (TP // SC_CHUNK)
    col = (wid % (TP // SC_CHUNK)) * SC_CHUNK
    pltpu.sync_copy(out_v, out_hbm.at[t, :, pl.ds(col, SC_CHUNK)])


def _build_masks(lab):
    mesh = plsc.VectorSubcoreMesh(core_axis_name="c", subcore_axis_name="s")
    return pl.kernel(
        _sc_mask_kernel,
        mesh=mesh,
        out_type=jax.ShapeDtypeStruct((NTP, MROWS, TP), jnp.float32),
        scratch_types=[
            pltpu.VMEM((N_ANCHOR, SC_CHUNK), jnp.int32),
            pltpu.VMEM((MROWS, SC_CHUNK), jnp.float32),
        ],
    )(lab)


def _fold8(v):
    w = v.shape[1]
    while w > 256:
        w //= 2
        v = v[:, :w] + v[:, w:]
    return v


def _dot_t(a, b):
    return lax.dot_general(a, b, (((1,), (1,)), ((), ())),
                           preferred_element_type=jnp.float32)


def _tc_body(x_ref, w_ref, m_ref, gb_ref, out_ref,
             ybf_s, asum, asq, macc, cnt, a_s, b_s):
    ph = pl.program_id(0)
    t = pl.program_id(1)

    @pl.when((ph == 0) & (t == 0))
    def _init():
        asum[...] = jnp.zeros_like(asum)
        asq[...] = jnp.zeros_like(asq)
        macc[...] = jnp.zeros_like(macc)
        cnt[...] = jnp.zeros_like(cnt)

    @pl.when(ph == 0)
    def _phase0():
        xbf = x_ref[...].astype(jnp.bfloat16)
        wbf = w_ref[...].astype(jnp.bfloat16)
        ybf = _dot_t(xbf, wbf).astype(jnp.bfloat16)
        ybf_s[pl.ds(t * TP, TP), :] = ybf
        o1 = jnp.ones((1, TP), jnp.bfloat16)
        asum[...] += jnp.dot(o1, ybf, preferred_element_type=jnp.float32)
        asq[...] += jnp.dot(o1, ybf * ybf, preferred_element_type=jnp.float32)

    @pl.when((ph == 0) & (t == NTP - 1))
    def _mid():
        inv_p = jnp.float32(1.0 / P_TOTAL)
        mean = asum[...] * inv_p
        var = asq[...] * inv_p - mean * mean
        a = gb_ref[0:1, :] * lax.rsqrt(var + jnp.float32(1e-5))
        a_s[...] = a
        b_s[...] = gb_ref[1:2, :] - a * mean

    @pl.when(ph == 1)
    def _phase1():
        yv = ybf_s[pl.ds(t * TP, TP), :]
        ab = a_s[...].astype(jnp.bfloat16)
        bb = b_s[...].astype(jnp.bfloat16)
        z = jnp.maximum(ab * yv + bb, jnp.bfloat16(0.0))
        zz = jnp.concatenate([z, z * z], axis=1)
        m2 = m_ref[0]
        macc[...] += jnp.dot(m2.astype(jnp.bfloat16), zz,
                             preferred_element_type=jnp.float32)
        cnt[...] += _fold8(m2)

    @pl.when((ph == 1) & (t == NTP - 1))
    def _fin():
        n_pos = jnp.sum(cnt[0:1, :])
        n_neg = jnp.sum(cnt[1:2, :])
        mu_p = macc[0:1, :C_OUT] / n_pos
        mu_n = macc[1:2, :C_OUT] / n_neg
        ss_p = macc[0:1, C_OUT:]
        ss_n = macc[1:2, C_OUT:]
        var_p = (ss_p - n_pos * mu_p * mu_p) / (n_pos - jnp.float32(1.0))
        var_n = (ss_n - n_neg * mu_n * mu_n) / (n_neg - jnp.float32(1.0))
        out_ref[...] = jnp.concatenate(
            [mu_p, mu_n, jnp.sqrt(var_p), jnp.sqrt(var_n)], axis=0)


def _tc_call(xt, w, masks, gb):
    return pl.pallas_call(
        _tc_body,
        grid=(2, NTP),
        in_specs=[
            pl.BlockSpec((TP, C_IN), lambda ph, t: (t * (1 - ph), 0)),
            pl.BlockSpec((C_OUT, C_IN), lambda ph, t: (0, 0)),
            pl.BlockSpec((1, MROWS, TP), lambda ph, t: (t, 0, 0)),
            pl.BlockSpec((2, C_OUT), lambda ph, t: (0, 0)),
        ],
        out_specs=pl.BlockSpec((4, C_OUT), lambda ph, t: (0, 0)),
        out_shape=jax.ShapeDtypeStruct((4, C_OUT), jnp.float32),
        scratch_shapes=[
            pltpu.VMEM((P_TOTAL, C_OUT), jnp.bfloat16),
            pltpu.VMEM((1, C_OUT), jnp.float32),
            pltpu.VMEM((1, C_OUT), jnp.float32),
            pltpu.VMEM((MROWS, 2 * C_OUT), jnp.float32),
            pltpu.VMEM((MROWS, C_OUT), jnp.float32),
            pltpu.VMEM((1, C_OUT), jnp.float32),
            pltpu.VMEM((1, C_OUT), jnp.float32),
        ],
    )(xt, w, masks, gb)


def kernel(hidden_map, label_cls, W, gamma, beta):
    s = hidden_map.shape
    xt = jnp.transpose(hidden_map, (0, 2, 3, 1)).reshape(P_TOTAL, C_IN)
    lab = label_cls.astype(jnp.int32).reshape(N_BATCH, N_ANCHOR, HW)
    masks = _build_masks(lab)
    gb = jnp.stack([gamma, beta])
    out = _tc_call(xt, W, masks, gb)
    latents = out.reshape(1, 4 * C_OUT, 1, 1)
    return (latents, 0.0, s)


# speedup vs baseline: 3.7163x; 1.0159x over previous
import jax
import jax.numpy as jnp
from jax import lax
from jax.experimental import pallas as pl
from jax.experimental.pallas import tpu as pltpu
from jax.experimental.pallas import tpu_sc as plsc

N_BATCH = 16
C_IN = 256
C_OUT = 256
HW = 4096
P_TOTAL = N_BATCH * HW
TP = 8192
NTP = P_TOTAL // TP
MROWS = 2
SC_CHUNK = 2048

SC_CORES = 2
SC_SUBCORES = 16
N_ANCHOR = 5
BLANES = 32


def _sc_mask_kernel(lab_hbm, out_hbm, lab_v, out_v):
    wid = lax.axis_index("s") * SC_CORES + lax.axis_index("c")
    n = wid // (HW // SC_CHUNK)
    off = (wid % (HW // SC_CHUNK)) * SC_CHUNK
    pltpu.sync_copy(lab_hbm.at[n, :, pl.ds(off, SC_CHUNK)], lab_v)

    def body(i, carry):
        base = pl.multiple_of(i * BLANES, BLANES)
        one = jnp.bfloat16(1.0)
        zero = jnp.bfloat16(0.0)
        acc_eq = jnp.zeros((BLANES,), jnp.bfloat16)
        acc_ge = jnp.zeros((BLANES,), jnp.bfloat16)
        for a_idx in range(N_ANCHOR):
            l = lab_v[a_idx, pl.ds(base, BLANES)]
            acc_eq += one - jnp.minimum(jnp.abs(l - one), one)
            acc_ge += one - jnp.minimum(jnp.maximum(-l, zero), one)
        pos = jnp.minimum(acc_eq, one)
        neg = jnp.minimum(acc_ge, one) * (one - pos)
        out_v[0, pl.ds(base, BLANES)] = pos
        out_v[1, pl.ds(base, BLANES)] = neg
        return carry

    lax.fori_loop(0, SC_CHUNK // BLANES, body, 0)
    t = wid // (TP // SC_CHUNK)
    col = (wid % (TP // SC_CHUNK)) * SC_CHUNK
    pltpu.sync_copy(out_v, out_hbm.at[t, :, pl.ds(col, SC_CHUNK)])


def _build_masks(lab):
    mesh = plsc.VectorSubcoreMesh(core_axis_name="c", subcore_axis_name="s")
    return pl.kernel(
        _sc_mask_kernel,
        mesh=mesh,
        out_type=jax.ShapeDtypeStruct((NTP, MROWS, TP), jnp.bfloat16),
        scratch_types=[
            pltpu.VMEM((N_ANCHOR, SC_CHUNK), jnp.bfloat16),
            pltpu.VMEM((MROWS, SC_CHUNK), jnp.bfloat16),
        ],
    )(lab)


def _fold(v):
    w = v.shape[1]
    while w > 256:
        w //= 2
        v = v[:, :w] + v[:, w:]
    return v


def _dot_t(a, b):
    return lax.dot_general(a, b, (((1,), (1,)), ((), ())),
                           preferred_element_type=jnp.float32)


def _tc_body(x_ref, w_ref, m_ref, gb_ref, out_ref,
             ybf_s, asum, asq, macc, cnt, a_s, b_s):
    ph = pl.program_id(0)
    t = pl.program_id(1)

    @pl.when((ph == 0) & (t == 0))
    def _init():
        asum[...] = jnp.zeros_like(asum)
        asq[...] = jnp.zeros_like(asq)
        macc[...] = jnp.zeros_like(macc)
        cnt[...] = jnp.zeros_like(cnt)

    @pl.when(ph == 0)
    def _phase0():
        xbf = x_ref[...].astype(jnp.bfloat16)
        wbf = w_ref[...].astype(jnp.bfloat16)
        ybf = _dot_t(xbf, wbf).astype(jnp.bfloat16)
        ybf_s[pl.ds(t * TP, TP), :] = ybf
        o1 = jnp.ones((1, TP), jnp.bfloat16)
        asum[...] += jnp.dot(o1, ybf, preferred_element_type=jnp.float32)
        asq[...] += jnp.dot(o1, ybf * ybf, preferred_element_type=jnp.float32)

    @pl.when((ph == 0) & (t == NTP - 1))
    def _mid():
        inv_p = jnp.float32(1.0 / P_TOTAL)
        mean = asum[...] * inv_p
        var = asq[...] * inv_p - mean * mean
        a = gb_ref[0:1, :] * lax.rsqrt(var + jnp.float32(1e-5))
        a_s[...] = a
        b_s[...] = gb_ref[1:2, :] - a * mean

    @pl.when(ph == 1)
    def _phase1():
        yv = ybf_s[pl.ds(t * TP, TP), :]
        ab = a_s[...].astype(jnp.bfloat16)
        bb = b_s[...].astype(jnp.bfloat16)
        z = jnp.maximum(ab * yv + bb, jnp.bfloat16(0.0))
        zz = jnp.concatenate([z, z * z], axis=1)
        m2 = m_ref[0]
        macc[...] += jnp.dot(m2, zz, preferred_element_type=jnp.float32)
        cnt[...] += _fold(m2).astype(jnp.float32)

    @pl.when((ph == 1) & (t == NTP - 1))
    def _fin():
        n_pos = jnp.sum(cnt[0:1, :])
        n_neg = jnp.sum(cnt[1:2, :])
        mu_p = macc[0:1, :C_OUT] / n_pos
        mu_n = macc[1:2, :C_OUT] / n_neg
        ss_p = macc[0:1, C_OUT:]
        ss_n = macc[1:2, C_OUT:]
        var_p = (ss_p - n_pos * mu_p * mu_p) / (n_pos - jnp.float32(1.0))
        var_n = (ss_n - n_neg * mu_n * mu_n) / (n_neg - jnp.float32(1.0))
        out_ref[...] = jnp.concatenate(
            [mu_p, mu_n, jnp.sqrt(var_p), jnp.sqrt(var_n)], axis=0)


def _tc_call(xt, w, masks, gb):
    return pl.pallas_call(
        _tc_body,
        grid=(2, NTP),
        in_specs=[
            pl.BlockSpec((TP, C_IN), lambda ph, t: (t * (1 - ph), 0)),
            pl.BlockSpec((C_OUT, C_IN), lambda ph, t: (0, 0)),
            pl.BlockSpec((1, MROWS, TP), lambda ph, t: (t, 0, 0)),
            pl.BlockSpec((2, C_OUT), lambda ph, t: (0, 0)),
        ],
        out_specs=pl.BlockSpec((4, C_OUT), lambda ph, t: (0, 0)),
        out_shape=jax.ShapeDtypeStruct((4, C_OUT), jnp.float32),
        scratch_shapes=[
            pltpu.VMEM((P_TOTAL, C_OUT), jnp.bfloat16),
            pltpu.VMEM((1, C_OUT), jnp.float32),
            pltpu.VMEM((1, C_OUT), jnp.float32),
            pltpu.VMEM((MROWS, 2 * C_OUT), jnp.float32),
            pltpu.VMEM((MROWS, C_OUT), jnp.float32),
            pltpu.VMEM((1, C_OUT), jnp.float32),
            pltpu.VMEM((1, C_OUT), jnp.float32),
        ],
    )(xt, w, masks, gb)


def kernel(hidden_map, label_cls, W, gamma, beta):
    s = hidden_map.shape
    xt = jnp.transpose(hidden_map, (0, 2, 3, 1)).reshape(P_TOTAL, C_IN)
    lab = label_cls.astype(jnp.bfloat16).reshape(N_BATCH, N_ANCHOR, HW)
    masks = _build_masks(lab)
    gb = jnp.stack([gamma, beta])
    out = _tc_call(xt, W, masks, gb)
    latents = out.reshape(1, 4 * C_OUT, 1, 1)
    return (latents, 0.0, s)
